# baseline (speedup 1.0000x reference)
import functools

import jax
import jax.numpy as jnp
from jax.experimental import pallas as pl
from jax.experimental.pallas import tpu as pltpu

_EPS = 1e-5
_VMEM_LIMIT = 48 * 1024 * 1024


def _block_body(xp_ref, s1_ref, b1_ref, w1_ref, s2_ref, b2_ref, w2_ref,
                wsc_ref, o_ref, *, ho, wo, cin, co):
    m = ho * wo
    f32 = jnp.float32

    a = xp_ref[...].astype(f32) * s1_ref[0] + b1_ref[0]
    a = jnp.maximum(a, 0.0).astype(jnp.bfloat16)

    pee = a[0, 0]
    poe = jnp.pad(a[1, 0], ((1, 0), (0, 0), (0, 0)))
    peo = jnp.pad(a[0, 1], ((0, 0), (1, 0), (0, 0)))
    poo = jnp.pad(a[1, 1], ((1, 0), (1, 0), (0, 0)))

    wins = (
        poo[0:ho, 0:wo], poe[0:ho, :], poo[0:ho, 1:wo + 1],
        peo[:, 0:wo], pee, peo[:, 1:wo + 1],
        poo[1:ho + 1, 0:wo], poe[1:ho + 1, :], poo[1:ho + 1, 1:wo + 1],
    )
    acc = jnp.zeros((m, co), f32)
    for t in range(9):
        acc = acc + jnp.dot(wins[t].reshape(m, cin), w1_ref[t],
                            preferred_element_type=f32)

    a2 = jnp.maximum(acc * s2_ref[0] + b2_ref[0], 0.0).astype(jnp.bfloat16)
    a2p = jnp.pad(a2.reshape(ho, wo, co), ((1, 1), (1, 1), (0, 0)))

    dims = (((1,), (1,)), ((), ()))
    out = jax.lax.dot_general(wsc_ref[...], pee.reshape(m, cin), dims,
                              preferred_element_type=f32)
    for t in range(9):
        dy, dx = divmod(t, 3)
        win = a2p[dy:dy + ho, dx:dx + wo].reshape(m, co)
        out = out + jax.lax.dot_general(w2_ref[t], win, dims,
                                        preferred_element_type=f32)

    o_ref[...] = out


def kernel(x, bn1_gamma, bn1_beta, bn1_mean, bn1_var,
           bn2_gamma, bn2_beta, bn2_mean, bn2_var, w1, w2, w_sc):
    n, cin, h, w = x.shape
    co = w1.shape[0]
    ho, wo = h // 2, w // 2
    m = ho * wo

    s1 = bn1_gamma / jnp.sqrt(bn1_var + _EPS)
    b1 = bn1_beta - bn1_mean * s1
    s2 = bn2_gamma / jnp.sqrt(bn2_var + _EPS)
    b2 = bn2_beta - bn2_mean * s2

    xp = x.transpose(0, 2, 3, 1).reshape(n, ho, 2, wo, 2, cin)
    xp = xp.transpose(0, 2, 4, 1, 3, 5).astype(jnp.bfloat16)

    wp1 = jnp.transpose(w1, (2, 3, 1, 0)).reshape(9, cin, co).astype(jnp.bfloat16)
    wp2 = jnp.transpose(w2, (2, 3, 0, 1)).reshape(9, co, co).astype(jnp.bfloat16)
    wsc = w_sc.reshape(co, cin).astype(jnp.bfloat16)

    body = functools.partial(_block_body, ho=ho, wo=wo, cin=cin, co=co)
    out = pl.pallas_call(
        body,
        grid=(n,),
        in_specs=[
            pl.BlockSpec((None, 2, 2, ho, wo, cin),
                         lambda i: (i, 0, 0, 0, 0, 0)),
            pl.BlockSpec((1, cin), lambda i: (0, 0)),
            pl.BlockSpec((1, cin), lambda i: (0, 0)),
            pl.BlockSpec((9, cin, co), lambda i: (0, 0, 0)),
            pl.BlockSpec((1, co), lambda i: (0, 0)),
            pl.BlockSpec((1, co), lambda i: (0, 0)),
            pl.BlockSpec((9, co, co), lambda i: (0, 0, 0)),
            pl.BlockSpec((co, cin), lambda i: (0, 0)),
        ],
        out_specs=pl.BlockSpec((None, co, m), lambda i: (i, 0, 0)),
        out_shape=jax.ShapeDtypeStruct((n, co, m), jnp.float32),
        compiler_params=pltpu.CompilerParams(
            dimension_semantics=("parallel",),
            vmem_limit_bytes=_VMEM_LIMIT),
        cost_estimate=pl.CostEstimate(
            flops=2 * n * m * 9 * (cin + co) * co + 2 * n * m * cin * co,
            transcendentals=0,
            bytes_accessed=2 * n * 4 * ho * wo * cin + 4 * n * m * co),
    )(xp, s1.reshape(1, cin), b1.reshape(1, cin), wp1,
      s2.reshape(1, co), b2.reshape(1, co), wp2, wsc)

    return out.reshape(n, co, ho, wo)


# speedup vs baseline: 1.7179x; 1.7179x over previous
import functools

import jax
import jax.numpy as jnp
from jax.experimental import pallas as pl
from jax.experimental.pallas import tpu as pltpu

_EPS = 1e-5
_VMEM_LIMIT = 48 * 1024 * 1024


def _one_image(a, s2, b2, w1_ref, w2_ref, wsc_ref, *, ho, wo, cin, co):
    m = ho * wo
    f32 = jnp.float32

    pee = a[0, 0]
    poe = jnp.pad(a[1, 0], ((1, 0), (0, 0), (0, 0)))
    peo = jnp.pad(a[0, 1], ((0, 0), (1, 0), (0, 0)))
    poo = jnp.pad(a[1, 1], ((1, 0), (1, 0), (0, 0)))

    wins = (
        poo[0:ho, 0:wo], poe[0:ho, :], poo[0:ho, 1:wo + 1],
        peo[:, 0:wo], pee, peo[:, 1:wo + 1],
        poo[1:ho + 1, 0:wo], poe[1:ho + 1, :], poo[1:ho + 1, 1:wo + 1],
    )
    acc = jnp.zeros((m, co), f32)
    for t in range(9):
        acc = acc + jnp.dot(wins[t].reshape(m, cin), w1_ref[t],
                            preferred_element_type=f32)

    a2 = jnp.maximum(acc * s2 + b2, 0.0).astype(jnp.bfloat16)
    a2p = jnp.pad(a2.reshape(ho, wo, co), ((1, 1), (1, 1), (0, 0)))

    out = jnp.dot(pee.reshape(m, cin), wsc_ref[...], preferred_element_type=f32)

    for t in range(9):
        dy, dx = divmod(t, 3)
        win = a2p[dy:dy + ho, dx:dx + wo].reshape(m, co)
        out = out + jnp.dot(win, w2_ref[t], preferred_element_type=f32)
    return out


def _block_body(xp_ref, s1_ref, b1_ref, w1_ref, s2_ref, b2_ref, w2_ref,
                wsc_ref, o_ref, *, nb, ho, wo, cin, co):
    m = ho * wo
    a = xp_ref[...].astype(jnp.float32) * s1_ref[0] + b1_ref[0]
    a = jnp.maximum(a, 0.0).astype(jnp.bfloat16)
    s2, b2 = s2_ref[0], b2_ref[0]
    for b in range(nb):
        o_ref[b * m:(b + 1) * m, :] = _one_image(
            a[b], s2, b2, w1_ref, w2_ref, wsc_ref,
            ho=ho, wo=wo, cin=cin, co=co)


def kernel(x, bn1_gamma, bn1_beta, bn1_mean, bn1_var,
           bn2_gamma, bn2_beta, bn2_mean, bn2_var, w1, w2, w_sc):
    n, cin, h, w = x.shape
    co = w1.shape[0]
    ho, wo = h // 2, w // 2
    m = ho * wo
    nb = 8 if n % 8 == 0 else 1

    s1 = bn1_gamma / jnp.sqrt(bn1_var + _EPS)
    b1 = bn1_beta - bn1_mean * s1
    s2 = bn2_gamma / jnp.sqrt(bn2_var + _EPS)
    b2 = bn2_beta - bn2_mean * s2

    xp = x.transpose(0, 2, 3, 1).reshape(n, ho, 2, wo, 2, cin)
    xp = xp.transpose(0, 2, 4, 1, 3, 5).astype(jnp.bfloat16)

    wp1 = jnp.transpose(w1, (2, 3, 1, 0)).reshape(9, cin, co).astype(jnp.bfloat16)
    wp2 = jnp.transpose(w2, (2, 3, 1, 0)).reshape(9, co, co).astype(jnp.bfloat16)
    wsc = jnp.transpose(w_sc.reshape(co, cin), (1, 0)).astype(jnp.bfloat16)

    body = functools.partial(_block_body, nb=nb, ho=ho, wo=wo, cin=cin, co=co)
    out = pl.pallas_call(
        body,
        grid=(n // nb,),
        in_specs=[
            pl.BlockSpec((nb, 2, 2, ho, wo, cin),
                         lambda i: (i, 0, 0, 0, 0, 0)),
            pl.BlockSpec((1, cin), lambda i: (0, 0)),
            pl.BlockSpec((1, cin), lambda i: (0, 0)),
            pl.BlockSpec((9, cin, co), lambda i: (0, 0, 0)),
            pl.BlockSpec((1, co), lambda i: (0, 0)),
            pl.BlockSpec((1, co), lambda i: (0, 0)),
            pl.BlockSpec((9, co, co), lambda i: (0, 0, 0)),
            pl.BlockSpec((cin, co), lambda i: (0, 0)),
        ],
        out_specs=pl.BlockSpec((nb * m, co), lambda i: (i, 0)),
        out_shape=jax.ShapeDtypeStruct((n * m, co), jnp.float32),
        compiler_params=pltpu.CompilerParams(
            dimension_semantics=("parallel",),
            vmem_limit_bytes=_VMEM_LIMIT),
        cost_estimate=pl.CostEstimate(
            flops=2 * n * m * 9 * (cin + co) * co + 2 * n * m * cin * co,
            transcendentals=0,
            bytes_accessed=2 * n * 4 * ho * wo * cin + 4 * n * m * co),
    )(xp, s1.reshape(1, cin), b1.reshape(1, cin), wp1,
      s2.reshape(1, co), b2.reshape(1, co), wp2, wsc)

    return out.reshape(n, ho, wo, co).transpose(0, 3, 1, 2)


# speedup vs baseline: 1.7780x; 1.0350x over previous
import functools

import jax
import jax.numpy as jnp
from jax.experimental import pallas as pl
from jax.experimental.pallas import tpu as pltpu

_EPS = 1e-5
_VMEM_LIMIT = 48 * 1024 * 1024


def _one_image(a_all, s2, b2, w1_ref, w2_ref, wsc_ref, *, ho, wo, cin, co):
    m = ho * wo
    f32 = jnp.float32

    ar = a_all.reshape(ho, 2, wo, 2 * cin)
    ev = ar[:, 0]
    od = ar[:, 1]

    p0 = jnp.concatenate([jnp.zeros((1, wo, 2 * cin), a_all.dtype),
                          od[:ho - 1]], axis=0)
    planes = (p0, ev, od)

    acc = jnp.zeros((m, co), f32)
    for dy in range(3):
        p = planes[dy]
        pl_shift = jnp.concatenate(
            [jnp.zeros((ho, 1, 2 * cin), a_all.dtype), p[:, :wo - 1]], axis=1)
        lhs = jnp.concatenate(
            [p.reshape(m, 2 * cin), pl_shift.reshape(m, 2 * cin)], axis=1)
        acc = acc + jnp.dot(lhs, w1_ref[dy], preferred_element_type=f32)

    a2 = jnp.maximum(acc * s2 + b2, 0.0).astype(jnp.bfloat16)
    a2p = jnp.pad(a2.reshape(ho, wo, co), ((1, 1), (1, 1), (0, 0)))

    out = jnp.dot(ev.reshape(m, 2 * cin), wsc_ref[...],
                  preferred_element_type=f32)

    for dy in range(3):
        rows = a2p[dy:dy + ho]
        lhs = jnp.concatenate([rows[:, 0:wo].reshape(m, co),
                               rows[:, 1:wo + 1].reshape(m, co),
                               rows[:, 2:wo + 2].reshape(m, co)], axis=1)
        out = out + jnp.dot(lhs, w2_ref[dy], preferred_element_type=f32)
    return out


def _block_body(x_ref, s1_ref, b1_ref, w1_ref, s2_ref, b2_ref, w2_ref,
                wsc_ref, o_ref, *, nb, ho, wo, cin, co):
    m = ho * wo
    a = x_ref[...].astype(jnp.float32) * s1_ref[0] + b1_ref[0]
    a = jnp.maximum(a, 0.0).astype(jnp.bfloat16)
    s2, b2 = s2_ref[0], b2_ref[0]
    for b in range(nb):
        o_ref[b * m:(b + 1) * m, :] = _one_image(
            a[b], s2, b2, w1_ref, w2_ref, wsc_ref,
            ho=ho, wo=wo, cin=cin, co=co)


def kernel(x, bn1_gamma, bn1_beta, bn1_mean, bn1_var,
           bn2_gamma, bn2_beta, bn2_mean, bn2_var, w1, w2, w_sc):
    n, cin, h, w = x.shape
    co = w1.shape[0]
    ho, wo = h // 2, w // 2
    m = ho * wo
    nb = 8 if n % 8 == 0 else 1
    bf16 = jnp.bfloat16

    s1 = bn1_gamma / jnp.sqrt(bn1_var + _EPS)
    b1 = bn1_beta - bn1_mean * s1
    s2 = bn2_gamma / jnp.sqrt(bn2_var + _EPS)
    b2 = bn2_beta - bn2_mean * s2

    xnh = x.transpose(0, 2, 3, 1).astype(bf16).reshape(n, h, wo, 2 * cin)

    s1t = jnp.concatenate([s1, s1]).reshape(1, 2 * cin).astype(jnp.float32)
    b1t = jnp.concatenate([b1, b1]).reshape(1, 2 * cin).astype(jnp.float32)

    zero = jnp.zeros((3, cin, co), jnp.float32)
    wt = jnp.transpose(w1, (2, 1, 3, 0))
    w1k = jnp.concatenate(
        [wt[:, :, 1], wt[:, :, 2], zero, wt[:, :, 0]], axis=1).astype(bf16)

    w2t = jnp.transpose(w2, (2, 1, 3, 0))
    w2k = jnp.concatenate(
        [w2t[:, :, 0], w2t[:, :, 1], w2t[:, :, 2]], axis=1).astype(bf16)

    wsc2 = jnp.concatenate(
        [w_sc.reshape(co, cin).T, jnp.zeros((cin, co))], axis=0).astype(bf16)

    body = functools.partial(_block_body, nb=nb, ho=ho, wo=wo, cin=cin, co=co)
    out = pl.pallas_call(
        body,
        grid=(n // nb,),
        in_specs=[
            pl.BlockSpec((nb, h, wo, 2 * cin), lambda i: (i, 0, 0, 0)),
            pl.BlockSpec((1, 2 * cin), lambda i: (0, 0)),
            pl.BlockSpec((1, 2 * cin), lambda i: (0, 0)),
            pl.BlockSpec((3, 4 * cin, co), lambda i: (0, 0, 0)),
            pl.BlockSpec((1, co), lambda i: (0, 0)),
            pl.BlockSpec((1, co), lambda i: (0, 0)),
            pl.BlockSpec((3, 3 * co, co), lambda i: (0, 0, 0)),
            pl.BlockSpec((2 * cin, co), lambda i: (0, 0)),
        ],
        out_specs=pl.BlockSpec((nb * m, co), lambda i: (i, 0)),
        out_shape=jax.ShapeDtypeStruct((n * m, co), jnp.float32),
        compiler_params=pltpu.CompilerParams(
            dimension_semantics=("parallel",),
            vmem_limit_bytes=_VMEM_LIMIT),
        cost_estimate=pl.CostEstimate(
            flops=2 * n * m * 9 * (cin + co) * co + 2 * n * m * cin * co,
            transcendentals=0,
            bytes_accessed=2 * n * h * w * cin + 4 * n * m * co),
    )(xnh, s1t, b1t, w1k, s2.reshape(1, co), b2.reshape(1, co), w2k, wsc2)

    return out.reshape(n, ho, wo, co).transpose(0, 3, 1, 2)


# speedup vs baseline: 1.9140x; 1.0765x over previous
import functools

import jax
import jax.numpy as jnp
from jax.experimental import pallas as pl
from jax.experimental.pallas import tpu as pltpu

_EPS = 1e-5
_VMEM_LIMIT = 48 * 1024 * 1024


def _one_image(a_all, s2, b2, w1_ref, w2_ref, o_ref, *, ho, wo, cin, co):
    m = ho * wo
    f32 = jnp.float32
    c2 = 2 * cin

    ar = a_all.reshape(ho, 2, wo, c2)
    ev = ar[:, 0]
    od = ar[:, 1]

    p0 = jnp.concatenate([jnp.zeros((1, wo, c2), a_all.dtype),
                          od[:ho - 1]], axis=0)
    planes = (p0, ev, od)

    pieces = []
    for dy in range(3):
        p = planes[dy]
        shift = jnp.concatenate(
            [jnp.zeros((ho, 1, c2), a_all.dtype), p[:, :wo - 1]], axis=1)
        pieces.append(p.reshape(m, c2))
        pieces.append(shift.reshape(m, c2))
    acc = jnp.dot(jnp.concatenate(pieces, axis=1), w1_ref[...],
                  preferred_element_type=f32)

    a2 = jnp.maximum(acc * s2 + b2, 0.0).astype(jnp.bfloat16)
    a2p = jnp.pad(a2.reshape(ho, wo, co), ((1, 1), (1, 1), (0, 0)))

    pieces = []
    for dy in range(3):
        rows = a2p[dy:dy + ho]
        for dx in range(3):
            pieces.append(rows[:, dx:dx + wo].reshape(m, co))
    pieces.append(ev.reshape(m, c2))
    out = jnp.dot(jnp.concatenate(pieces, axis=1), w2_ref[...],
                  preferred_element_type=f32)
    o_ref[...] = out.reshape(ho, wo, co)


def _block_body(x_ref, s1_ref, b1_ref, w1_ref, s2_ref, b2_ref, w2_ref,
                o_ref, *, nb, ho, wo, cin, co):
    xi = pltpu.bitcast(x_ref[...], jnp.int32)
    lo = jax.lax.bitcast_convert_type(xi.astype(jnp.int16), jnp.bfloat16)
    hi = jax.lax.bitcast_convert_type(
        (xi >> 16).astype(jnp.int16), jnp.bfloat16)
    xp = jnp.concatenate([lo, hi], axis=-1)

    a = xp.astype(jnp.float32) * s1_ref[0] + b1_ref[0]
    a = jnp.maximum(a, 0.0).astype(jnp.bfloat16)
    s2, b2 = s2_ref[0], b2_ref[0]
    for b in range(nb):
        _one_image(a[b], s2, b2, w1_ref, w2_ref, o_ref.at[b],
                   ho=ho, wo=wo, cin=cin, co=co)


def kernel(x, bn1_gamma, bn1_beta, bn1_mean, bn1_var,
           bn2_gamma, bn2_beta, bn2_mean, bn2_var, w1, w2, w_sc):
    n, cin, h, w = x.shape
    co = w1.shape[0]
    ho, wo = h // 2, w // 2
    nb = 8 if n % 8 == 0 else 1
    bf16 = jnp.bfloat16

    s1 = bn1_gamma / jnp.sqrt(bn1_var + _EPS)
    b1 = bn1_beta - bn1_mean * s1
    s2 = bn2_gamma / jnp.sqrt(bn2_var + _EPS)
    b2 = bn2_beta - bn2_mean * s2

    xnh = x.transpose(0, 2, 3, 1).astype(bf16)

    s1t = jnp.concatenate([s1, s1]).reshape(1, 2 * cin).astype(jnp.float32)
    b1t = jnp.concatenate([b1, b1]).reshape(1, 2 * cin).astype(jnp.float32)

    zero = jnp.zeros((3, cin, co), jnp.float32)
    wt = jnp.transpose(w1, (2, 1, 3, 0))
    w1k = jnp.concatenate(
        [wt[:, :, 1], wt[:, :, 2], zero, wt[:, :, 0]],
        axis=1).reshape(3 * 4 * cin, co).astype(bf16)

    w2t = jnp.transpose(w2, (2, 3, 1, 0)).reshape(9 * co, co)
    w2k = jnp.concatenate(
        [w2t, w_sc.reshape(co, cin).T, jnp.zeros((cin, co))],
        axis=0).astype(bf16)

    body = functools.partial(_block_body, nb=nb, ho=ho, wo=wo, cin=cin, co=co)
    out = pl.pallas_call(
        body,
        grid=(n // nb,),
        in_specs=[
            pl.BlockSpec((nb, h, w, cin), lambda i: (i, 0, 0, 0)),
            pl.BlockSpec((1, 2 * cin), lambda i: (0, 0)),
            pl.BlockSpec((1, 2 * cin), lambda i: (0, 0)),
            pl.BlockSpec((12 * cin, co), lambda i: (0, 0)),
            pl.BlockSpec((1, co), lambda i: (0, 0)),
            pl.BlockSpec((1, co), lambda i: (0, 0)),
            pl.BlockSpec((9 * co + 2 * cin, co), lambda i: (0, 0)),
        ],
        out_specs=pl.BlockSpec((nb, ho, wo, co), lambda i: (i, 0, 0, 0)),
        out_shape=jax.ShapeDtypeStruct((n, ho, wo, co), jnp.float32),
        compiler_params=pltpu.CompilerParams(
            dimension_semantics=("parallel",),
            vmem_limit_bytes=_VMEM_LIMIT),
        cost_estimate=pl.CostEstimate(
            flops=2 * n * ho * wo * 9 * (cin + co) * co,
            transcendentals=0,
            bytes_accessed=2 * n * h * w * cin + 4 * n * ho * wo * co),
    )(xnh, s1t, b1t, w1k, s2.reshape(1, co), b2.reshape(1, co), w2k)

    return jnp.transpose(out, (0, 3, 1, 2))


# speedup vs baseline: 1.9177x; 1.0019x over previous
import functools

import jax
import jax.numpy as jnp
from jax.experimental import pallas as pl
from jax.experimental.pallas import tpu as pltpu

_EPS = 1e-5
_VMEM_LIMIT = 48 * 1024 * 1024


def _one_image(a_all, s2, b2, w1_ref, w2_ref, o_ref, *, ho, wo, cin, co):
    m = ho * wo
    f32 = jnp.float32
    c2 = 2 * cin

    ar = a_all.reshape(ho, 2, wo, c2)
    ev = ar[:, 0]
    od = ar[:, 1]

    p0 = jnp.concatenate([jnp.zeros((1, wo, c2), a_all.dtype),
                          od[:ho - 1]], axis=0)
    planes = (p0, ev, od)

    pieces = []
    for dy in range(3):
        p = planes[dy]
        shift = jnp.concatenate(
            [jnp.zeros((ho, 1, c2), a_all.dtype), p[:, :wo - 1]], axis=1)
        pieces.append(p.reshape(m, c2))
        pieces.append(shift.reshape(m, c2))
    acc = jnp.dot(jnp.concatenate(pieces, axis=1), w1_ref[...],
                  preferred_element_type=f32)

    a2 = jnp.maximum(acc * s2 + b2, 0.0).astype(jnp.bfloat16)
    a2p = jnp.pad(a2.reshape(ho, wo, co), ((1, 1), (1, 1), (0, 0)))

    pieces = []
    for dy in range(3):
        rows = a2p[dy:dy + ho]
        for dx in range(3):
            pieces.append(rows[:, dx:dx + wo].reshape(m, co))
    pieces.append(ev.reshape(m, c2))
    out = jnp.dot(jnp.concatenate(pieces, axis=1), w2_ref[...],
                  preferred_element_type=f32)
    o_ref[...] = out.reshape(ho, wo, co)


def _block_body(x_ref, s1_ref, b1_ref, w1_ref, s2_ref, b2_ref, w2_ref,
                o_ref, *, nb, ho, wo, cin, co):
    xi = pltpu.bitcast(x_ref[...], jnp.int32)
    lo = jax.lax.bitcast_convert_type(xi.astype(jnp.int16), jnp.bfloat16)
    hi = jax.lax.bitcast_convert_type(
        (xi >> 16).astype(jnp.int16), jnp.bfloat16)
    xp = jnp.concatenate([lo, hi], axis=-1)

    a = xp.astype(jnp.float32) * s1_ref[0] + b1_ref[0]
    a = jnp.maximum(a, 0.0).astype(jnp.bfloat16)
    s2, b2 = s2_ref[0], b2_ref[0]
    for b in range(nb):
        _one_image(a[b], s2, b2, w1_ref, w2_ref, o_ref.at[b],
                   ho=ho, wo=wo, cin=cin, co=co)


def kernel(x, bn1_gamma, bn1_beta, bn1_mean, bn1_var,
           bn2_gamma, bn2_beta, bn2_mean, bn2_var, w1, w2, w_sc):
    n, cin, h, w = x.shape
    co = w1.shape[0]
    ho, wo = h // 2, w // 2
    nb = 16 if n % 16 == 0 else 1
    bf16 = jnp.bfloat16

    s1 = bn1_gamma / jnp.sqrt(bn1_var + _EPS)
    b1 = bn1_beta - bn1_mean * s1
    s2 = bn2_gamma / jnp.sqrt(bn2_var + _EPS)
    b2 = bn2_beta - bn2_mean * s2

    xnh = x.transpose(0, 2, 3, 1).astype(bf16)

    s1t = jnp.concatenate([s1, s1]).reshape(1, 2 * cin).astype(jnp.float32)
    b1t = jnp.concatenate([b1, b1]).reshape(1, 2 * cin).astype(jnp.float32)

    zero = jnp.zeros((3, cin, co), jnp.float32)
    wt = jnp.transpose(w1, (2, 1, 3, 0))
    w1k = jnp.concatenate(
        [wt[:, :, 1], wt[:, :, 2], zero, wt[:, :, 0]],
        axis=1).reshape(3 * 4 * cin, co).astype(bf16)

    w2t = jnp.transpose(w2, (2, 3, 1, 0)).reshape(9 * co, co)
    w2k = jnp.concatenate(
        [w2t, w_sc.reshape(co, cin).T, jnp.zeros((cin, co))],
        axis=0).astype(bf16)

    body = functools.partial(_block_body, nb=nb, ho=ho, wo=wo, cin=cin, co=co)
    out = pl.pallas_call(
        body,
        grid=(n // nb,),
        in_specs=[
            pl.BlockSpec((nb, h, w, cin), lambda i: (i, 0, 0, 0)),
            pl.BlockSpec((1, 2 * cin), lambda i: (0, 0)),
            pl.BlockSpec((1, 2 * cin), lambda i: (0, 0)),
            pl.BlockSpec((12 * cin, co), lambda i: (0, 0)),
            pl.BlockSpec((1, co), lambda i: (0, 0)),
            pl.BlockSpec((1, co), lambda i: (0, 0)),
            pl.BlockSpec((9 * co + 2 * cin, co), lambda i: (0, 0)),
        ],
        out_specs=pl.BlockSpec((nb, ho, wo, co), lambda i: (i, 0, 0, 0)),
        out_shape=jax.ShapeDtypeStruct((n, ho, wo, co), jnp.float32),
        compiler_params=pltpu.CompilerParams(
            dimension_semantics=("parallel",),
            vmem_limit_bytes=_VMEM_LIMIT),
        cost_estimate=pl.CostEstimate(
            flops=2 * n * ho * wo * 9 * (cin + co) * co,
            transcendentals=0,
            bytes_accessed=2 * n * h * w * cin + 4 * n * ho * wo * co),
    )(xnh, s1t, b1t, w1k, s2.reshape(1, co), b2.reshape(1, co), w2k)

    return jnp.transpose(out, (0, 3, 1, 2))


# speedup vs baseline: 1.9244x; 1.0035x over previous
import functools

import jax
import jax.numpy as jnp
from jax.experimental import pallas as pl
from jax.experimental.pallas import tpu as pltpu

_EPS = 1e-5
_VMEM_LIMIT = 48 * 1024 * 1024


def _one_image(a_all, s2, b2, w1_ref, w2_ref, o_ref, *, ho, wo, cin, co):
    m = ho * wo
    f32 = jnp.float32
    c2 = 2 * cin

    ar = a_all.reshape(ho, 2, wo, c2)
    ev = ar[:, 0]
    od = ar[:, 1]

    p0 = jnp.concatenate([jnp.zeros((1, wo, c2), a_all.dtype),
                          od[:ho - 1]], axis=0)
    planes = (p0, ev, od)

    pieces = []
    for dy in range(3):
        p = planes[dy]
        shift = jnp.concatenate(
            [jnp.zeros((ho, 1, c2), a_all.dtype), p[:, :wo - 1]], axis=1)
        pieces.append(p.reshape(m, c2))
        pieces.append(shift.reshape(m, c2))
    acc = jnp.dot(jnp.concatenate(pieces, axis=1), w1_ref[...],
                  preferred_element_type=f32)

    a2 = jnp.maximum(acc * s2 + b2, 0.0).astype(jnp.bfloat16)
    a2p = jnp.pad(a2.reshape(ho, wo, co), ((1, 1), (1, 1), (0, 0)))

    pieces = []
    for dy in range(3):
        rows = a2p[dy:dy + ho]
        for dx in range(3):
            pieces.append(rows[:, dx:dx + wo].reshape(m, co))
    pieces.append(ev.reshape(m, c2))
    out = jnp.dot(jnp.concatenate(pieces, axis=1), w2_ref[...],
                  preferred_element_type=f32)
    o_ref[...] = out.reshape(ho, wo, co)


def _block_body(x_ref, b1_ref, w1_ref, s2_ref, b2_ref, w2_ref,
                o_ref, *, nb, ho, wo, cin, co):
    xi = pltpu.bitcast(x_ref[...], jnp.int32)
    lo = jax.lax.bitcast_convert_type(xi.astype(jnp.int16), jnp.bfloat16)
    hi = jax.lax.bitcast_convert_type(
        (xi >> 16).astype(jnp.int16), jnp.bfloat16)
    xp = jnp.concatenate([lo, hi], axis=-1)

    a = jnp.maximum(xp.astype(jnp.float32) + b1_ref[0], 0.0).astype(jnp.bfloat16)
    s2, b2 = s2_ref[0], b2_ref[0]
    for b in range(nb):
        _one_image(a[b], s2, b2, w1_ref, w2_ref, o_ref.at[b],
                   ho=ho, wo=wo, cin=cin, co=co)


def kernel(x, bn1_gamma, bn1_beta, bn1_mean, bn1_var,
           bn2_gamma, bn2_beta, bn2_mean, bn2_var, w1, w2, w_sc):
    n, cin, h, w = x.shape
    co = w1.shape[0]
    ho, wo = h // 2, w // 2
    nb = 16 if n % 16 == 0 else 1
    bf16 = jnp.bfloat16

    s1 = bn1_gamma / jnp.sqrt(bn1_var + _EPS)
    b1 = bn1_beta - bn1_mean * s1
    s2 = bn2_gamma / jnp.sqrt(bn2_var + _EPS)
    b2 = bn2_beta - bn2_mean * s2

    xnh = x.transpose(0, 2, 3, 1).astype(bf16)

    b1s = b1 / s1
    b1t = jnp.concatenate([b1s, b1s]).reshape(1, 2 * cin).astype(jnp.float32)

    zero = jnp.zeros((3, cin, co), jnp.float32)
    wt = jnp.transpose(w1, (2, 1, 3, 0)) * s1[None, :, None, None]
    w1k = jnp.concatenate(
        [wt[:, :, 1], wt[:, :, 2], zero, wt[:, :, 0]],
        axis=1).reshape(3 * 4 * cin, co).astype(bf16)

    w2t = jnp.transpose(w2, (2, 3, 1, 0)).reshape(9 * co, co)
    w2k = jnp.concatenate(
        [w2t, w_sc.reshape(co, cin).T * s1[:, None], jnp.zeros((cin, co))],
        axis=0).astype(bf16)

    body = functools.partial(_block_body, nb=nb, ho=ho, wo=wo, cin=cin, co=co)
    out = pl.pallas_call(
        body,
        grid=(n // nb,),
        in_specs=[
            pl.BlockSpec((nb, h, w, cin), lambda i: (i, 0, 0, 0)),
            pl.BlockSpec((1, 2 * cin), lambda i: (0, 0)),
            pl.BlockSpec((12 * cin, co), lambda i: (0, 0)),
            pl.BlockSpec((1, co), lambda i: (0, 0)),
            pl.BlockSpec((1, co), lambda i: (0, 0)),
            pl.BlockSpec((9 * co + 2 * cin, co), lambda i: (0, 0)),
        ],
        out_specs=pl.BlockSpec((nb, ho, wo, co), lambda i: (i, 0, 0, 0)),
        out_shape=jax.ShapeDtypeStruct((n, ho, wo, co), jnp.float32),
        compiler_params=pltpu.CompilerParams(
            dimension_semantics=("parallel",),
            vmem_limit_bytes=_VMEM_LIMIT),
        cost_estimate=pl.CostEstimate(
            flops=2 * n * ho * wo * 9 * (cin + co) * co,
            transcendentals=0,
            bytes_accessed=2 * n * h * w * cin + 4 * n * ho * wo * co),
    )(xnh, b1t, w1k, s2.reshape(1, co), b2.reshape(1, co), w2k)

    return jnp.transpose(out, (0, 3, 1, 2))
